# baseline (speedup 1.0000x reference)
import functools

import jax
import jax.numpy as jnp
from jax.experimental import pallas as pl
from jax.experimental.pallas import tpu as pltpu


def _pos_kernel(col_ref, row_ref, out_hbm, pos_v, sems, *, B, H, W, D):
    HW = H * W
    ce = col_ref[0:W, :]
    re = row_ref[0:H, :]

    row_w = jax.lax.broadcasted_iota(jnp.int32, (W, HW), 0)
    lane_w = jax.lax.broadcasted_iota(jnp.int32, (W, HW), 1)
    sel_w = (lane_w % W == row_w).astype(jnp.float32)

    row_h = jax.lax.broadcasted_iota(jnp.int32, (H, HW), 0)
    lane_h = jax.lax.broadcasted_iota(jnp.int32, (H, HW), 1)
    sel_h = (lane_h // W == row_h).astype(jnp.float32)

    dims = (((0,), (0,)), ((), ()))
    pos_v[0:D, :] = jax.lax.dot_general(
        ce, sel_w, dims, precision=jax.lax.Precision.HIGHEST,
        preferred_element_type=jnp.float32)
    pos_v[D:2 * D, :] = jax.lax.dot_general(
        re, sel_h, dims, precision=jax.lax.Precision.HIGHEST,
        preferred_element_type=jnp.float32)

    n_sems = sems.shape[0]
    copies = []
    for b in range(B):
        copies.append(
            pltpu.make_async_copy(pos_v, out_hbm.at[b], sems.at[b % n_sems]))
    for cp in copies:
        cp.start()
    for cp in copies:
        cp.wait()


def kernel(x, row_embed, col_embed):
    B, C, H, W = x.shape
    D = row_embed.shape[1]
    HW = H * W

    body = functools.partial(_pos_kernel, B=B, H=H, W=W, D=D)

    out = pl.pallas_call(
        body,
        in_specs=[
            pl.BlockSpec(memory_space=pltpu.VMEM),
            pl.BlockSpec(memory_space=pltpu.VMEM),
        ],
        out_specs=pl.BlockSpec(memory_space=pl.ANY),
        out_shape=jax.ShapeDtypeStruct((B, 2 * D, HW), jnp.float32),
        scratch_shapes=[
            pltpu.VMEM((2 * D, HW), jnp.float32),
            pltpu.SemaphoreType.DMA((4,)),
        ],
    )(col_embed, row_embed)
    return out.reshape(B, 2 * D, H, W)
